# Initial kernel scaffold; baseline (speedup 1.0000x reference)
import jax
import jax.numpy as jnp
from jax.experimental import pallas as pl


def kernel(X, edge_index, edge_vals, W, b):
    raise NotImplementedError("write your pallas kernel here")


# speedup vs baseline: 2.9006x; 2.9006x over previous
import functools

import jax
import jax.numpy as jnp
from jax import lax
from jax.experimental import pallas as pl
from jax.experimental.pallas import tpu as pltpu
from jax.experimental.pallas import tpu_sc as plsc

N = 10000
E = 320000
D = 128

NC = 2
NS = 16
NW = NC * NS

N_PAD = 10240
E_PAD = 327680
EPW = E_PAD // NW
K = 128
CHUNKS = EPW // K


def _matmul_body(x_ref, wt_ref, b_ref, h_ref):
    h_ref[...] = (
        jnp.dot(x_ref[...], wt_ref[...], preferred_element_type=jnp.float32)
        + b_ref[0:1, :]
    )


def _linear(X, Wt, b2):
    blk = 400
    return pl.pallas_call(
        _matmul_body,
        grid=(N // blk,),
        in_specs=[
            pl.BlockSpec((blk, D), lambda i: (i, 0)),
            pl.BlockSpec((D, D), lambda i: (0, 0)),
            pl.BlockSpec((8, D), lambda i: (0, 0)),
        ],
        out_specs=pl.BlockSpec((blk, D), lambda i: (i, 0)),
        out_shape=jax.ShapeDtypeStruct((N, D), jnp.float32),
    )(X, Wt, b2)


def _spmm_body(h_hbm, rows_hbm, cols_hbm, vals_hbm, out_hbm,
               acc, rows_v, colv, rowv, valv, gsem):
    cid = lax.axis_index("c")
    sid = lax.axis_index("s")

    zero16 = jnp.zeros((16,), jnp.float32)

    def zero_row(j, _):
        for q in range(D // 16):
            rows_v[j, pl.ds(q * 16, 16)] = zero16
        return 0

    lax.fori_loop(0, K, zero_row, 0)
    slab = sid * (N_PAD // NS)
    for t in range(N_PAD // NS // K):
        pltpu.sync_copy(rows_v, acc.at[pl.ds(slab + t * K, K)])
    plsc.subcore_barrier()

    base = (cid * NS + sid) * EPW

    def chunk_body(c, _):
        off = base + c * K
        pltpu.sync_copy(cols_hbm.at[pl.ds(off, K)], colv)
        gather = pltpu.async_copy(h_hbm.at[colv], rows_v, gsem)
        pltpu.sync_copy(vals_hbm.at[pl.ds(off, K)], valv)
        pltpu.sync_copy(rows_hbm.at[pl.ds(off, K)], rowv)
        gather.wait()

        def scale_grp(jj, _):
            ev = valv[pl.ds(jj * 16, 16)]
            for e in range(16):
                s = ev[e]
                j = jj * 16 + e
                for q in range(D // 16):
                    sl = pl.ds(q * 16, 16)
                    rows_v[j, sl] = rows_v[j, sl] * s
            return 0

        lax.fori_loop(0, K // 16, scale_grp, 0)
        pltpu.sync_copy(rows_v, acc.at[rowv], add=True)
        return 0

    lax.fori_loop(0, CHUNKS, chunk_body, 0)
    plsc.subcore_barrier()

    pltpu.sync_copy(acc.at[pl.ds(slab, N_PAD // NS)],
                    out_hbm.at[cid, pl.ds(slab, N_PAD // NS)])


def _spmm(H, rows, cols, vals):
    mesh = plsc.VectorSubcoreMesh(
        core_axis_name="c", subcore_axis_name="s", num_cores=NC,
        num_subcores=NS)
    return pl.kernel(
        _spmm_body,
        out_type=jax.ShapeDtypeStruct((NC, N_PAD, D), jnp.float32),
        mesh=mesh,
        scratch_types=[
            pltpu.VMEM_SHARED((N_PAD, D), jnp.float32),
            pltpu.VMEM((K, D), jnp.float32),
            pltpu.VMEM((K,), jnp.int32),
            pltpu.VMEM((K,), jnp.int32),
            pltpu.VMEM((K,), jnp.float32),
            pltpu.SemaphoreType.DMA,
        ],
    )(H, rows, cols, vals)


def _combine_body(p0_ref, p1_ref, o_ref):
    o_ref[...] = jnp.maximum(p0_ref[0] + p1_ref[0], 0.0)


def _combine(P):
    blk = 320
    return pl.pallas_call(
        _combine_body,
        grid=(N_PAD // blk,),
        in_specs=[
            pl.BlockSpec((1, blk, D), lambda i: (0, i, 0)),
            pl.BlockSpec((1, blk, D), lambda i: (1, i, 0)),
        ],
        out_specs=pl.BlockSpec((blk, D), lambda i: (i, 0)),
        out_shape=jax.ShapeDtypeStruct((N_PAD, D), jnp.float32),
    )(P, P)


def kernel(X, edge_index, edge_vals, W, b):
    Wt = W.T
    b2 = jnp.broadcast_to(b, (8, D))
    H = _linear(X, Wt, b2)

    pad = E_PAD - E
    rows = jnp.concatenate([edge_index[0], jnp.zeros((pad,), jnp.int32)])
    cols = jnp.concatenate([edge_index[1], jnp.zeros((pad,), jnp.int32)])
    vals = jnp.concatenate([edge_vals, jnp.zeros((pad,), jnp.float32)])

    P = _spmm(H, rows, cols, vals)
    out = _combine(P)
    return out[:N]


# speedup vs baseline: 3.6276x; 1.2506x over previous
import functools

import jax
import jax.numpy as jnp
from jax import lax
from jax.experimental import pallas as pl
from jax.experimental.pallas import tpu as pltpu
from jax.experimental.pallas import tpu_sc as plsc

N = 10000
E = 320000
D = 128

NC = 2
NS = 16
NW = NC * NS

N_PAD = 10240
E_PAD = 327680
EPW = E_PAD // NW
K = 128
CHUNKS = EPW // K


def _matmul_body(x_ref, wt_ref, b_ref, h_ref):
    h_ref[...] = (
        jnp.dot(x_ref[...], wt_ref[...], preferred_element_type=jnp.float32)
        + b_ref[0:1, :]
    )


def _linear(X, Wt, b2):
    blk = 400
    return pl.pallas_call(
        _matmul_body,
        grid=(N // blk,),
        in_specs=[
            pl.BlockSpec((blk, D), lambda i: (i, 0)),
            pl.BlockSpec((D, D), lambda i: (0, 0)),
            pl.BlockSpec((8, D), lambda i: (0, 0)),
        ],
        out_specs=pl.BlockSpec((blk, D), lambda i: (i, 0)),
        out_shape=jax.ShapeDtypeStruct((N, D), jnp.float32),
    )(X, Wt, b2)


def _spmm_body(h_hbm, rows_hbm, cols_hbm, vals_hbm, out_hbm,
               acc, r0, r1, c0, c1, w0, w1, v0, v1,
               g0, g1, s0, s1):
    cid = lax.axis_index("c")
    sid = lax.axis_index("s")
    bufs = ((r0, c0, w0, v0, g0, s0), (r1, c1, w1, v1, g1, s1))

    zero16 = jnp.zeros((16,), jnp.float32)

    def zero_row(j, _):
        for q in range(D // 16):
            r0[j, pl.ds(q * 16, 16)] = zero16
        return 0

    lax.fori_loop(0, K, zero_row, 0)
    slab = sid * (N_PAD // NS)
    for t in range(N_PAD // NS // K):
        pltpu.sync_copy(r0, acc.at[pl.ds(slab + t * K, K)])
    plsc.subcore_barrier()

    base = (cid * NS + sid) * EPW

    def load_and_gather(c, buf):
        rbuf, cbuf, wbuf, vbuf, gsem, _ = buf
        off = base + c * K
        pltpu.sync_copy(cols_hbm.at[pl.ds(off, K)], cbuf)
        pltpu.async_copy(h_hbm.at[cbuf], rbuf, gsem)
        pltpu.sync_copy(vals_hbm.at[pl.ds(off, K)], vbuf)
        pltpu.sync_copy(rows_hbm.at[pl.ds(off, K)], wbuf)

    load_and_gather(0, bufs[0])

    def pair(p, _):
        for b in range(2):
            c = 2 * p + b
            cur = bufs[b]
            nxt = bufs[1 - b]
            rbuf, cbuf, wbuf, vbuf, gsem, ssem = cur

            @pl.when(c > 0)
            def _():
                pltpu.make_async_copy(nxt[0], acc.at[nxt[2]], nxt[5]).wait()

            load_and_gather(c + 1, nxt)
            pltpu.make_async_copy(h_hbm.at[cbuf], rbuf, gsem).wait()

            def scale_grp(jj, _):
                ev = vbuf[pl.ds(jj * 16, 16)]
                for e in range(16):
                    s = ev[e]
                    j = jj * 16 + e
                    for q in range(D // 16):
                        sl = pl.ds(q * 16, 16)
                        rbuf[j, sl] = rbuf[j, sl] * s
                return 0

            lax.fori_loop(0, K // 16, scale_grp, 0)
            pltpu.async_copy(rbuf, acc.at[wbuf], ssem, add=True)
        return 0

    lax.fori_loop(0, CHUNKS // 2, pair, 0)
    pltpu.make_async_copy(r1, acc.at[w1], s1).wait()
    pltpu.make_async_copy(h_hbm.at[c0], r0, g0).wait()
    plsc.subcore_barrier()

    pltpu.sync_copy(acc.at[pl.ds(slab, N_PAD // NS)],
                    out_hbm.at[cid, pl.ds(slab, N_PAD // NS)])


def _spmm(H, rows, cols, vals):
    mesh = plsc.VectorSubcoreMesh(
        core_axis_name="c", subcore_axis_name="s", num_cores=NC,
        num_subcores=NS)
    return pl.kernel(
        _spmm_body,
        out_type=jax.ShapeDtypeStruct((NC, N_PAD, D), jnp.float32),
        mesh=mesh,
        scratch_types=[
            pltpu.VMEM_SHARED((N_PAD, D), jnp.float32),
            pltpu.VMEM((K, D), jnp.float32),
            pltpu.VMEM((K, D), jnp.float32),
            pltpu.VMEM((K,), jnp.int32),
            pltpu.VMEM((K,), jnp.int32),
            pltpu.VMEM((K,), jnp.int32),
            pltpu.VMEM((K,), jnp.int32),
            pltpu.VMEM((K,), jnp.float32),
            pltpu.VMEM((K,), jnp.float32),
            pltpu.SemaphoreType.DMA,
            pltpu.SemaphoreType.DMA,
            pltpu.SemaphoreType.DMA,
            pltpu.SemaphoreType.DMA,
        ],
    )(H, rows, cols, vals)


def _combine_body(p0_ref, p1_ref, o_ref):
    o_ref[...] = jnp.maximum(p0_ref[0] + p1_ref[0], 0.0)


def _combine(P):
    blk = 320
    return pl.pallas_call(
        _combine_body,
        grid=(N_PAD // blk,),
        in_specs=[
            pl.BlockSpec((1, blk, D), lambda i: (0, i, 0)),
            pl.BlockSpec((1, blk, D), lambda i: (1, i, 0)),
        ],
        out_specs=pl.BlockSpec((blk, D), lambda i: (i, 0)),
        out_shape=jax.ShapeDtypeStruct((N_PAD, D), jnp.float32),
    )(P, P)


def kernel(X, edge_index, edge_vals, W, b):
    Wt = W.T
    b2 = jnp.broadcast_to(b, (8, D))
    H = _linear(X, Wt, b2)

    pad = E_PAD + K - E
    rows = jnp.concatenate([edge_index[0], jnp.zeros((pad,), jnp.int32)])
    cols = jnp.concatenate([edge_index[1], jnp.zeros((pad,), jnp.int32)])
    vals = jnp.concatenate([edge_vals, jnp.zeros((pad,), jnp.float32)])

    P = _spmm(H, rows, cols, vals)
    out = _combine(P)
    return out[:N]
